# baseline (speedup 1.0000x reference)
import jax
import jax.numpy as jnp
from jax.experimental import pallas as pl

D = 2048
BATCH = 16384
BM = 512
KC = 512


def _table_kernel(tt_ref, w1_ref, b_ref, t_ref):
    k = pl.program_id(0)
    part = jax.lax.dot_general(
        tt_ref[...], w1_ref[...], (((1,), (1,)), ((), ())),
        preferred_element_type=jnp.float32)

    @pl.when(k == 0)
    def _():
        t_ref[...] = part + b_ref[...]

    @pl.when(k > 0)
    def _():
        t_ref[...] += part


def _main_kernel(emb_ref, add_ref, w2_ref, out_ref):
    acc = jax.lax.dot_general(
        emb_ref[...], w2_ref[...], (((1,), (1,)), ((), ())),
        preferred_element_type=jnp.float32)
    out_ref[...] = acc + add_ref[...]


def kernel(embedding, task_idxs, task_table, W, b):
    n = W.shape[0]
    nt = task_table.shape[0]
    t = pl.pallas_call(
        _table_kernel,
        grid=(D // KC,),
        in_specs=[
            pl.BlockSpec((nt, KC), lambda k: (0, k)),
            pl.BlockSpec((n, KC), lambda k: (0, k)),
            pl.BlockSpec((1, n), lambda k: (0, 0)),
        ],
        out_specs=pl.BlockSpec((nt, n), lambda k: (0, 0)),
        out_shape=jax.ShapeDtypeStruct((nt, n), jnp.float32),
    )(task_table, W, b.reshape(1, n))

    addend = jnp.take(t, task_idxs.astype(jnp.int32), axis=0)

    grid = (BATCH // BM,)
    out = pl.pallas_call(
        _main_kernel,
        grid=grid,
        in_specs=[
            pl.BlockSpec((BM, D), lambda i: (i, 0)),
            pl.BlockSpec((BM, n), lambda i: (i, 0)),
            pl.BlockSpec((n, D), lambda i: (0, 1)),
        ],
        out_specs=pl.BlockSpec((BM, n), lambda i: (i, 0)),
        out_shape=jax.ShapeDtypeStruct((BATCH, n), jnp.float32),
    )(embedding, addend, W)
    return out
